# baseline (speedup 1.0000x reference)
import functools

import jax
import jax.numpy as jnp
from jax import lax
from jax.experimental import pallas as pl
from jax.experimental.pallas import tpu as pltpu
from jax.experimental.pallas import tpu_sc as plsc

N = 50000
NP = 50176
D = 90
HC = 64
KTOP = 70
G = 64
GP = 80
E = 800000
CH = 128
SUP = 3
EPR = 6432
EPAD = (EPR + SUP) * CH
BATP = 53248
RB = 512
KEYPAD = 67584
CB = 16384
RSEG = 112


@functools.lru_cache(maxsize=1)
def _sc_mesh():
    return plsc.VectorSubcoreMesh(core_axis_name="c", subcore_axis_name="s",
                                  num_cores=2, num_subcores=16)


def _zero_vmem2d(ref, rows, cols):
    z16 = jnp.zeros((16,), jnp.float32)
    def body(i, _):
        for cj in range(cols // 16):
            ref[i, pl.ds(cj * 16, 16)] = z16
        return 0
    lax.fori_loop(0, rows, body, 0)


def _agg_body(w_cols, src2, dst2, t0, t1, out, acc,
              ixs0, ixd0, ixs1, ixd1, rows0, rows1, stage,
              semg0, semg1, sems0, sems1):
    c = lax.axis_index("c")
    s = lax.axis_index("s")
    base = s * (NP // 16)
    rpt = EPR // 16
    nsup = rpt // SUP
    tile0 = s * rpt

    _zero_vmem2d(stage, RSEG, w_cols)
    for i in range(28):
        pltpu.sync_copy(stage, acc.at[pl.ds(base + i * RSEG, RSEG), :])
    plsc.subcore_barrier()

    def run(tbl):
        def load_idx(ixs, ixd, sup):
            r0 = tile0 + sup * SUP
            pltpu.sync_copy(src2.at[pl.ds(r0, SUP)], ixs)
            pltpu.sync_copy(dst2.at[pl.ds(r0, SUP)], ixd)

        def fire_g(ixs, rows, semg):
            for j in range(SUP):
                pltpu.async_copy(tbl.at[ixs.at[j]], rows.at[j], semg)

        def wait_g(ixs, rows, semg):
            for j in range(SUP):
                pltpu.make_async_copy(tbl.at[ixs.at[j]], rows.at[j],
                                      semg).wait()

        def fire_s(ixd, rows, sems):
            for j in range(SUP):
                pltpu.async_copy(rows.at[j], acc.at[ixd.at[j]], sems,
                                 add=True)

        def wait_s(ixd, rows, sems):
            for j in range(SUP):
                pltpu.make_async_copy(rows.at[j], acc.at[ixd.at[j]],
                                      sems).wait()

        load_idx(ixs0, ixd0, jnp.int32(0))
        fire_g(ixs0, rows0, semg0)

        def pair_body(i, _):
            wait_g(ixs0, rows0, semg0)
            fire_s(ixd0, rows0, sems0)
            @pl.when(i > 0)
            def _():
                wait_s(ixd1, rows1, sems1)
            load_idx(ixs1, ixd1, 2 * i + 1)
            fire_g(ixs1, rows1, semg1)
            wait_g(ixs1, rows1, semg1)
            fire_s(ixd1, rows1, sems1)
            wait_s(ixd0, rows0, sems0)
            load_idx(ixs0, ixd0, 2 * i + 2)
            fire_g(ixs0, rows0, semg0)
            return 0
        lax.fori_loop(0, nsup // 2, pair_body, 0)
        wait_s(ixd1, rows1, sems1)
        wait_g(ixs0, rows0, semg0)

    @pl.when(c == 0)
    def _():
        run(t0)

    @pl.when(c == 1)
    def _():
        run(t1)

    plsc.subcore_barrier()
    for i in range(28):
        pltpu.sync_copy(acc.at[pl.ds(base + i * RSEG, RSEG), :], stage)
        pltpu.sync_copy(stage, out.at[c, pl.ds(base + i * RSEG, RSEG), :])


@functools.lru_cache(maxsize=None)
def _agg_call(w_cols):
    return pl.kernel(
        functools.partial(_agg_body, w_cols),
        out_type=jax.ShapeDtypeStruct((2, NP, w_cols), jnp.float32),
        mesh=_sc_mesh(),
        compiler_params=pltpu.CompilerParams(use_tc_tiling_on_sc=False,
                                             needs_layout_passes=False),
        scratch_types=[
            pltpu.VMEM_SHARED((NP, w_cols), jnp.float32),
            pltpu.VMEM((SUP, CH), jnp.int32),
            pltpu.VMEM((SUP, CH), jnp.int32),
            pltpu.VMEM((SUP, CH), jnp.int32),
            pltpu.VMEM((SUP, CH), jnp.int32),
            pltpu.VMEM((SUP, CH, w_cols), jnp.float32),
            pltpu.VMEM((SUP, CH, w_cols), jnp.float32),
            pltpu.VMEM((RSEG, w_cols), jnp.float32),
            pltpu.SemaphoreType.DMA,
            pltpu.SemaphoreType.DMA,
            pltpu.SemaphoreType.DMA,
            pltpu.SemaphoreType.DMA,
        ],
    )


def _cnt_body(dst2, bat2, outc, outg, accc, accg, idx_d, ones_v, stage, sem):
    c = lax.axis_index("c")
    s = lax.axis_index("s")
    w = c * 16 + s
    base = s * (NP // 16)

    one16 = jnp.ones((16,), jnp.float32)
    def fill_ones(i, _):
        ones_v[i, pl.ds(0, 16)] = one16
        return 0
    lax.fori_loop(0, CH, fill_ones, 0)

    _zero_vmem2d(stage, RSEG, 16)
    for i in range(28):
        pltpu.sync_copy(stage, accc.at[pl.ds(base + i * RSEG, RSEG), :])
    @pl.when(s == 0)
    def _():
        pltpu.sync_copy(stage.at[pl.ds(0, GP), :], accg)
    plsc.subcore_barrier()

    def cnt_sc(i, _):
        row0 = w * (EPR // 32) + i * SUP
        pltpu.sync_copy(dst2.at[pl.ds(row0, SUP)], idx_d)
        for j in range(SUP):
            pltpu.sync_copy(ones_v, accc.at[idx_d.at[j]], add=True)
        return 0
    lax.fori_loop(0, EPR // 32 // SUP, cnt_sc, 0)

    def bat_sc(i, _):
        row0 = w * (BATP // 128 // 32) + i
        pltpu.sync_copy(bat2.at[pl.ds(row0, 1)], idx_d.at[pl.ds(0, 1)])
        pltpu.sync_copy(ones_v, accg.at[idx_d.at[0]], add=True)
        return 0
    lax.fori_loop(0, BATP // 128 // 32, bat_sc, 0)

    plsc.subcore_barrier()
    for i in range(28):
        pltpu.sync_copy(accc.at[pl.ds(base + i * RSEG, RSEG), :], stage)
        pltpu.sync_copy(stage, outc.at[c, pl.ds(base + i * RSEG, RSEG), :])
    @pl.when(s == 0)
    def _():
        pltpu.sync_copy(accg, stage.at[pl.ds(0, GP), :])
        pltpu.sync_copy(stage.at[pl.ds(0, GP), :], outg.at[c])


@functools.lru_cache(maxsize=1)
def _cnt_call():
    return pl.kernel(
        _cnt_body,
        out_type=(jax.ShapeDtypeStruct((2, NP, 16), jnp.float32),
                  jax.ShapeDtypeStruct((2, GP, 16), jnp.float32)),
        mesh=_sc_mesh(),
        compiler_params=pltpu.CompilerParams(use_tc_tiling_on_sc=False,
                                             needs_layout_passes=False),
        scratch_types=[
            pltpu.VMEM_SHARED((NP, 16), jnp.float32),
            pltpu.VMEM_SHARED((GP, 16), jnp.float32),
            pltpu.VMEM((SUP, CH), jnp.int32),
            pltpu.VMEM((CH, 16), jnp.float32),
            pltpu.VMEM((RSEG, 16), jnp.float32),
            pltpu.SemaphoreType.DMA,
        ],
    )


def _pool_body(keyp, h2, hg0, hg1, pool, keys_v, rows_v, sel_v, h0_v, h1_v,
               st_s, cn_s, sem):
    c = lax.axis_index("c")
    s = lax.axis_index("s")
    w = c * 16 + s

    pltpu.sync_copy(hg0, h0_v)
    pltpu.sync_copy(hg1, h1_v)

    lanes = lax.iota(jnp.int32, 16)
    NEG = jnp.float32(-jnp.inf)
    BIG = jnp.int32(2**31 - 1)

    run = jnp.int32(0)
    for g in range(G):
        cg = (h0_v[g, pl.ds(0, 16)][0]
              + h1_v[g, pl.ds(0, 16)][0]).astype(jnp.int32)
        st_s[g] = run
        cn_s[g] = cg
        run = run + cg

    z16i = jnp.zeros((16,), jnp.int32)
    for o in (0, 16, 32, 48, KTOP + 2 - 16):
        sel_v[pl.ds(o, 16)] = z16i

    for gi in range(2):
        g = w * 2 + gi
        start = st_s[g]
        cnt = cn_s[g]
        abase = (start // 8) * 8
        off = start - abase
        nch = (cnt + (CB - 1)) // CB

        def load_chunk(ch):
            pltpu.sync_copy(keyp.at[pl.ds(abase + ch * CB, CB + 16)], keys_v)

        @pl.when(nch <= 1)
        def _():
            load_chunk(jnp.int32(0))

        def round_body(r, th):
            tk, ti = th

            def chunk_body(ch, bb):
                bk, bi = bb
                @pl.when(nch > 1)
                def _():
                    load_chunk(ch)
                glend = jnp.minimum((ch + 1) * CB, cnt)
                jtrip = (off + (glend - ch * CB) + 15) // 16

                def scan_body(j, bb2):
                    bk2, bi2 = bb2
                    k16 = keys_v[pl.ds(j * 16, 16)]
                    gl = ch * CB + j * 16 + lanes - off
                    ok = (gl >= ch * CB) & (gl < glend)
                    elig = ok & ((k16 < tk) | ((k16 == tk) & (gl > ti)))
                    better = (k16 > bk2) | ((k16 == bk2) & (gl < bi2))
                    upd = elig & better
                    return (jnp.where(upd, k16, bk2), jnp.where(upd, gl, bi2))

                return lax.fori_loop(0, jtrip, scan_body, (bk, bi))

            bk0 = jnp.full((16,), NEG, jnp.float32)
            bi0 = jnp.full((16,), BIG, jnp.int32)
            bk, bi = lax.fori_loop(0, nch, chunk_body, (bk0, bi0))
            m = jnp.max(bk)
            il = jnp.min(jnp.where(bk == m, bi, BIG))
            val = jnp.where(r < cnt, start + il, 0) + z16i
            plsc.store_scatter(sel_v, [r + z16i], val, mask=lanes == 0)
            return (m, il)

        lax.fori_loop(0, KTOP, round_body, (jnp.float32(jnp.inf),
                                            jnp.int32(-1)))

        pltpu.async_copy(h2.at[sel_v], rows_v, sem).wait()
        zz = jnp.zeros((16,), jnp.float32)
        def mask_body(r, _):
            ok = r < cnt
            for cj in range(HC // 16):
                v = rows_v[r, pl.ds(cj * 16, 16)]
                rows_v[r, pl.ds(cj * 16, 16)] = jnp.where(ok, v, zz)
            return 0
        lax.fori_loop(0, KTOP, mask_body, 0)
        pltpu.sync_copy(rows_v.at[pl.ds(0, KTOP), :], pool.at[g])


@functools.lru_cache(maxsize=1)
def _pool_call():
    return pl.kernel(
        _pool_body,
        out_type=jax.ShapeDtypeStruct((G, KTOP, HC), jnp.float32),
        mesh=_sc_mesh(),
        compiler_params=pltpu.CompilerParams(use_tc_tiling_on_sc=False,
                                             needs_layout_passes=False),
        scratch_types=[
            pltpu.VMEM((CB + 16,), jnp.float32),
            pltpu.VMEM((KTOP + 2, HC), jnp.float32),
            pltpu.VMEM((KTOP + 2,), jnp.int32),
            pltpu.VMEM((GP, 16), jnp.float32),
            pltpu.VMEM((GP, 16), jnp.float32),
            pltpu.SMEM((GP,), jnp.int32),
            pltpu.SMEM((GP,), jnp.int32),
            pltpu.SemaphoreType.DMA,
        ],
    )


def _n1_body(a0_ref, a1_ref, a2_ref, a3_ref, c0_ref, c1_ref, x_ref, wl_ref,
             wr_ref, b1_ref, xt_ref, h0_ref, h1_ref):
    cnt = jnp.maximum(c0_ref[:, 0:1] + c1_ref[:, 0:1], 1.0)
    mean = jnp.concatenate(
        [a0_ref[...], a1_ref[...], a2_ref[...], a3_ref[...]], axis=1)[:, :D]
    mean = mean / cnt
    xt = (jnp.dot(mean, wl_ref[...], preferred_element_type=jnp.float32)
          + b1_ref[...]) + jnp.dot(x_ref[...], wr_ref[...],
                                   preferred_element_type=jnp.float32)
    xt_ref[...] = xt
    h = jnp.maximum(xt, 0.0)
    h0_ref[...] = h[:, :32]
    h1_ref[...] = h[:, 32:]


def _n2_body(a0_ref, a1_ref, c0_ref, c1_ref, xt_ref, wl_ref, wr_ref, b2_ref,
             h2_ref, key_ref):
    cnt = jnp.maximum(c0_ref[:, 0:1] + c1_ref[:, 0:1], 1.0)
    mean = jnp.concatenate([a0_ref[...], a1_ref[...]], axis=1) / cnt
    h = jnp.maximum(xt_ref[...], 0.0)
    h2 = (jnp.dot(mean, wl_ref[...], preferred_element_type=jnp.float32)
          + b2_ref[...]) + jnp.dot(h, wr_ref[...],
                                   preferred_element_type=jnp.float32)
    h2_ref[...] = h2
    key_ref[...] = h2[:, HC - 1:HC]


def _f_body(p_ref, w1_ref, b1_ref, w2_ref, b2_ref, o_ref):
    p1 = (jnp.dot(p_ref[...], w1_ref[...], preferred_element_type=jnp.float32)
          + b1_ref[...])
    p2 = (jnp.dot(p1, w2_ref[...], preferred_element_type=jnp.float32)
          + b2_ref[...])
    o_ref[...] = jax.nn.sigmoid(p2)


def _row_block(shape):
    return pl.BlockSpec((RB,) + shape[1:],
                        lambda i: (i,) + (0,) * (len(shape) - 1))


def _full(shape):
    return pl.BlockSpec(shape, lambda i: (0,) * len(shape))


def kernel(x, edge_index, edge_weight, batch, W1l, b1, W1r, W2l, b2, W2r,
           Wlin1, blin1, Wlin2, blin2):
    del edge_weight
    f32 = jnp.float32
    nb = NP // RB

    xp = jnp.pad(x, ((0, NP - N), (0, 0)))
    src2 = jnp.pad(edge_index[0], (0, EPAD - E), constant_values=N).reshape(
        EPAD // 128, 128)
    dst2 = jnp.pad(edge_index[1], (0, EPAD - E), constant_values=N).reshape(
        EPAD // 128, 128)
    bat2 = jnp.pad(batch, (0, BATP - N), constant_values=G).reshape(
        BATP // 128, 128)

    x0 = xp[:, 0:32]
    x1 = xp[:, 32:64]
    x2 = xp[:, 64:80]
    x3 = jnp.pad(xp[:, 80:90], ((0, 0), (0, 6)))

    cntp, histp = _cnt_call()(dst2, bat2)
    agga = _agg_call(32)(src2, dst2, x0, x1)
    aggb = _agg_call(16)(src2, dst2, x2, x3)

    xt, h0, h1 = pl.pallas_call(
        _n1_body,
        grid=(nb,),
        in_specs=[_row_block((NP, 32)), _row_block((NP, 32)),
                  _row_block((NP, 16)), _row_block((NP, 16)),
                  _row_block((NP, 16)), _row_block((NP, 16)),
                  _row_block((NP, D)), _full((D, HC)), _full((D, HC)),
                  _full((1, HC))],
        out_specs=[_row_block((NP, HC)), _row_block((NP, 32)),
                   _row_block((NP, 32))],
        out_shape=[jax.ShapeDtypeStruct((NP, HC), f32),
                   jax.ShapeDtypeStruct((NP, 32), f32),
                   jax.ShapeDtypeStruct((NP, 32), f32)],
    )(agga[0], agga[1], aggb[0], aggb[1], cntp[0], cntp[1], xp, W1l, W1r,
      b1.reshape(1, HC))

    agg2 = _agg_call(32)(src2, dst2, h0, h1)

    h2, key = pl.pallas_call(
        _n2_body,
        grid=(nb,),
        in_specs=[_row_block((NP, 32)), _row_block((NP, 32)),
                  _row_block((NP, 16)), _row_block((NP, 16)),
                  _row_block((NP, HC)), _full((HC, HC)), _full((HC, HC)),
                  _full((1, HC))],
        out_specs=[_row_block((NP, HC)), _row_block((NP, 1))],
        out_shape=[jax.ShapeDtypeStruct((NP, HC), f32),
                   jax.ShapeDtypeStruct((NP, 1), f32)],
    )(agg2[0], agg2[1], cntp[0], cntp[1], xt, W2l, W2r, b2.reshape(1, HC))

    keyp = jnp.pad(key.reshape(NP), (0, KEYPAD - NP))
    pool = _pool_call()(keyp, h2, histp[0], histp[1])

    out = pl.pallas_call(
        _f_body,
        grid=(1,),
        in_specs=[_full((G, KTOP * HC)), _full((KTOP * HC, HC)),
                  _full((1, HC)), _full((HC, 1)), _full((1, 1))],
        out_specs=_full((G, 1)),
        out_shape=jax.ShapeDtypeStruct((G, 1), f32),
    )(pool.reshape(G, KTOP * HC), Wlin1, blin1.reshape(1, HC),
      Wlin2, blin2.reshape(1, 1))

    return (out.reshape(G), xt[:N])


# speedup vs baseline: 1.0409x; 1.0409x over previous
import functools

import jax
import jax.numpy as jnp
from jax import lax
from jax.experimental import pallas as pl
from jax.experimental.pallas import tpu as pltpu
from jax.experimental.pallas import tpu_sc as plsc

N = 50000
NP = 50176
D = 90
HC = 64
KTOP = 70
G = 64
GP = 80
E = 800000
CH = 128
SUP = 3
SUPA = 6
EPR = 6432
EPAD = (EPR + SUP) * CH
BATP = 53248
RB = 512
KEYPAD = 67584
CB = 16384
RSEG = 112


@functools.lru_cache(maxsize=1)
def _sc_mesh():
    return plsc.VectorSubcoreMesh(core_axis_name="c", subcore_axis_name="s",
                                  num_cores=2, num_subcores=16)


def _zero_vmem2d(ref, rows, cols):
    z16 = jnp.zeros((16,), jnp.float32)
    def body(i, _):
        for cj in range(cols // 16):
            ref[i, pl.ds(cj * 16, 16)] = z16
        return 0
    lax.fori_loop(0, rows, body, 0)


def _agg_body(w_cols, src2, dst2, t0, t1, out, acc,
              ixs0, ixd0, rows0, stage, semg0, sems0):
    c = lax.axis_index("c")
    s = lax.axis_index("s")
    base = s * (NP // 16)
    rpt = EPR // 16
    tile0 = s * rpt

    _zero_vmem2d(stage, RSEG, w_cols)
    for i in range(28):
        pltpu.sync_copy(stage, acc.at[pl.ds(base + i * RSEG, RSEG), :])
    plsc.subcore_barrier()

    def run(tbl):
        def sc_body(i, _):
            r0 = tile0 + i * SUPA
            pltpu.sync_copy(src2.at[pl.ds(r0, SUPA)], ixs0)
            pltpu.sync_copy(dst2.at[pl.ds(r0, SUPA)], ixd0)
            cps = [pltpu.async_copy(tbl.at[ixs0.at[j]], rows0.at[j], semg0)
                   for j in range(SUPA)]
            for cp in cps:
                cp.wait()
            cps = [pltpu.async_copy(rows0.at[j], acc.at[ixd0.at[j]], sems0,
                                    add=True) for j in range(SUPA)]
            for cp in cps:
                cp.wait()
            return 0
        lax.fori_loop(0, rpt // SUPA, sc_body, 0)

    @pl.when(c == 0)
    def _():
        run(t0)

    @pl.when(c == 1)
    def _():
        run(t1)

    plsc.subcore_barrier()
    for i in range(28):
        pltpu.sync_copy(acc.at[pl.ds(base + i * RSEG, RSEG), :], stage)
        pltpu.sync_copy(stage, out.at[c, pl.ds(base + i * RSEG, RSEG), :])


@functools.lru_cache(maxsize=None)
def _agg_call(w_cols):
    return pl.kernel(
        functools.partial(_agg_body, w_cols),
        out_type=jax.ShapeDtypeStruct((2, NP, w_cols), jnp.float32),
        mesh=_sc_mesh(),
        compiler_params=pltpu.CompilerParams(use_tc_tiling_on_sc=False,
                                             needs_layout_passes=False),
        scratch_types=[
            pltpu.VMEM_SHARED((NP, w_cols), jnp.float32),
            pltpu.VMEM((SUPA, CH), jnp.int32),
            pltpu.VMEM((SUPA, CH), jnp.int32),
            pltpu.VMEM((SUPA, CH, w_cols), jnp.float32),
            pltpu.VMEM((RSEG, w_cols), jnp.float32),
            pltpu.SemaphoreType.DMA,
            pltpu.SemaphoreType.DMA,
        ],
    )


def _cnt_body(dst2, bat2, outc, outg, accc, accg, idx_d, ones_v, stage, sem):
    c = lax.axis_index("c")
    s = lax.axis_index("s")
    w = c * 16 + s
    base = s * (NP // 16)

    one16 = jnp.ones((16,), jnp.float32)
    def fill_ones(i, _):
        ones_v[i, pl.ds(0, 16)] = one16
        return 0
    lax.fori_loop(0, CH, fill_ones, 0)

    _zero_vmem2d(stage, RSEG, 16)
    for i in range(28):
        pltpu.sync_copy(stage, accc.at[pl.ds(base + i * RSEG, RSEG), :])
    @pl.when(s == 0)
    def _():
        pltpu.sync_copy(stage.at[pl.ds(0, GP), :], accg)
    plsc.subcore_barrier()

    def cnt_sc(i, _):
        row0 = w * (EPR // 32) + i * SUP
        pltpu.sync_copy(dst2.at[pl.ds(row0, SUP)], idx_d)
        for j in range(SUP):
            pltpu.sync_copy(ones_v, accc.at[idx_d.at[j]], add=True)
        return 0
    lax.fori_loop(0, EPR // 32 // SUP, cnt_sc, 0)

    def bat_sc(i, _):
        row0 = w * (BATP // 128 // 32) + i
        pltpu.sync_copy(bat2.at[pl.ds(row0, 1)], idx_d.at[pl.ds(0, 1)])
        pltpu.sync_copy(ones_v, accg.at[idx_d.at[0]], add=True)
        return 0
    lax.fori_loop(0, BATP // 128 // 32, bat_sc, 0)

    plsc.subcore_barrier()
    for i in range(28):
        pltpu.sync_copy(accc.at[pl.ds(base + i * RSEG, RSEG), :], stage)
        pltpu.sync_copy(stage, outc.at[c, pl.ds(base + i * RSEG, RSEG), :])
    @pl.when(s == 0)
    def _():
        pltpu.sync_copy(accg, stage.at[pl.ds(0, GP), :])
        pltpu.sync_copy(stage.at[pl.ds(0, GP), :], outg.at[c])


@functools.lru_cache(maxsize=1)
def _cnt_call():
    return pl.kernel(
        _cnt_body,
        out_type=(jax.ShapeDtypeStruct((2, NP, 16), jnp.float32),
                  jax.ShapeDtypeStruct((2, GP, 16), jnp.float32)),
        mesh=_sc_mesh(),
        compiler_params=pltpu.CompilerParams(use_tc_tiling_on_sc=False,
                                             needs_layout_passes=False),
        scratch_types=[
            pltpu.VMEM_SHARED((NP, 16), jnp.float32),
            pltpu.VMEM_SHARED((GP, 16), jnp.float32),
            pltpu.VMEM((SUP, CH), jnp.int32),
            pltpu.VMEM((CH, 16), jnp.float32),
            pltpu.VMEM((RSEG, 16), jnp.float32),
            pltpu.SemaphoreType.DMA,
        ],
    )


def _pool_body(keyp, h2, hg0, hg1, pool, keys_v, rows_v, sel_v, h0_v, h1_v,
               st_s, cn_s, sem):
    c = lax.axis_index("c")
    s = lax.axis_index("s")
    w = c * 16 + s

    pltpu.sync_copy(hg0, h0_v)
    pltpu.sync_copy(hg1, h1_v)

    lanes = lax.iota(jnp.int32, 16)
    NEG = jnp.float32(-jnp.inf)
    BIG = jnp.int32(2**31 - 1)

    run = jnp.int32(0)
    for g in range(G):
        cg = (h0_v[g, pl.ds(0, 16)][0]
              + h1_v[g, pl.ds(0, 16)][0]).astype(jnp.int32)
        st_s[g] = run
        cn_s[g] = cg
        run = run + cg

    z16i = jnp.zeros((16,), jnp.int32)
    for o in (0, 16, 32, 48, KTOP + 2 - 16):
        sel_v[pl.ds(o, 16)] = z16i

    for gi in range(2):
        g = w * 2 + gi
        start = st_s[g]
        cnt = cn_s[g]
        abase = (start // 8) * 8
        off = start - abase
        nch = (cnt + (CB - 1)) // CB

        def load_chunk(ch):
            pltpu.sync_copy(keyp.at[pl.ds(abase + ch * CB, CB + 16)], keys_v)

        @pl.when(nch <= 1)
        def _():
            load_chunk(jnp.int32(0))

        def round_body(r, th):
            tk, ti = th

            def chunk_body(ch, bb):
                bk, bi = bb
                @pl.when(nch > 1)
                def _():
                    load_chunk(ch)
                glend = jnp.minimum((ch + 1) * CB, cnt)
                jtrip = (off + (glend - ch * CB) + 15) // 16

                def scan_body(j, bb2):
                    bk2, bi2 = bb2
                    k16 = keys_v[pl.ds(j * 16, 16)]
                    gl = ch * CB + j * 16 + lanes - off
                    ok = (gl >= ch * CB) & (gl < glend)
                    elig = ok & ((k16 < tk) | ((k16 == tk) & (gl > ti)))
                    better = (k16 > bk2) | ((k16 == bk2) & (gl < bi2))
                    upd = elig & better
                    return (jnp.where(upd, k16, bk2), jnp.where(upd, gl, bi2))

                return lax.fori_loop(0, jtrip, scan_body, (bk, bi))

            bk0 = jnp.full((16,), NEG, jnp.float32)
            bi0 = jnp.full((16,), BIG, jnp.int32)
            bk, bi = lax.fori_loop(0, nch, chunk_body, (bk0, bi0))
            m = jnp.max(bk)
            il = jnp.min(jnp.where(bk == m, bi, BIG))
            val = jnp.where(r < cnt, start + il, 0) + z16i
            plsc.store_scatter(sel_v, [r + z16i], val, mask=lanes == 0)
            return (m, il)

        lax.fori_loop(0, KTOP, round_body, (jnp.float32(jnp.inf),
                                            jnp.int32(-1)))

        pltpu.async_copy(h2.at[sel_v], rows_v, sem).wait()
        zz = jnp.zeros((16,), jnp.float32)
        def mask_body(r, _):
            ok = r < cnt
            for cj in range(HC // 16):
                v = rows_v[r, pl.ds(cj * 16, 16)]
                rows_v[r, pl.ds(cj * 16, 16)] = jnp.where(ok, v, zz)
            return 0
        lax.fori_loop(0, KTOP, mask_body, 0)
        pltpu.sync_copy(rows_v.at[pl.ds(0, KTOP), :], pool.at[g])


@functools.lru_cache(maxsize=1)
def _pool_call():
    return pl.kernel(
        _pool_body,
        out_type=jax.ShapeDtypeStruct((G, KTOP, HC), jnp.float32),
        mesh=_sc_mesh(),
        compiler_params=pltpu.CompilerParams(use_tc_tiling_on_sc=False,
                                             needs_layout_passes=False),
        scratch_types=[
            pltpu.VMEM((CB + 16,), jnp.float32),
            pltpu.VMEM((KTOP + 2, HC), jnp.float32),
            pltpu.VMEM((KTOP + 2,), jnp.int32),
            pltpu.VMEM((GP, 16), jnp.float32),
            pltpu.VMEM((GP, 16), jnp.float32),
            pltpu.SMEM((GP,), jnp.int32),
            pltpu.SMEM((GP,), jnp.int32),
            pltpu.SemaphoreType.DMA,
        ],
    )


def _n1_body(a0_ref, a1_ref, a2_ref, a3_ref, c0_ref, c1_ref, x_ref, wl_ref,
             wr_ref, b1_ref, xt_ref, h0_ref, h1_ref):
    cnt = jnp.maximum(c0_ref[:, 0:1] + c1_ref[:, 0:1], 1.0)
    mean = jnp.concatenate(
        [a0_ref[...], a1_ref[...], a2_ref[...], a3_ref[...]], axis=1)[:, :D]
    mean = mean / cnt
    xt = (jnp.dot(mean, wl_ref[...], preferred_element_type=jnp.float32)
          + b1_ref[...]) + jnp.dot(x_ref[...], wr_ref[...],
                                   preferred_element_type=jnp.float32)
    xt_ref[...] = xt
    h = jnp.maximum(xt, 0.0)
    h0_ref[...] = h[:, :32]
    h1_ref[...] = h[:, 32:]


def _n2_body(a0_ref, a1_ref, c0_ref, c1_ref, xt_ref, wl_ref, wr_ref, b2_ref,
             h2_ref, key_ref):
    cnt = jnp.maximum(c0_ref[:, 0:1] + c1_ref[:, 0:1], 1.0)
    mean = jnp.concatenate([a0_ref[...], a1_ref[...]], axis=1) / cnt
    h = jnp.maximum(xt_ref[...], 0.0)
    h2 = (jnp.dot(mean, wl_ref[...], preferred_element_type=jnp.float32)
          + b2_ref[...]) + jnp.dot(h, wr_ref[...],
                                   preferred_element_type=jnp.float32)
    h2_ref[...] = h2
    key_ref[...] = h2[:, HC - 1:HC]


def _f_body(p_ref, w1_ref, b1_ref, w2_ref, b2_ref, o_ref):
    p1 = (jnp.dot(p_ref[...], w1_ref[...], preferred_element_type=jnp.float32)
          + b1_ref[...])
    p2 = (jnp.dot(p1, w2_ref[...], preferred_element_type=jnp.float32)
          + b2_ref[...])
    o_ref[...] = jax.nn.sigmoid(p2)


def _row_block(shape):
    return pl.BlockSpec((RB,) + shape[1:],
                        lambda i: (i,) + (0,) * (len(shape) - 1))


def _full(shape):
    return pl.BlockSpec(shape, lambda i: (0,) * len(shape))


def kernel(x, edge_index, edge_weight, batch, W1l, b1, W1r, W2l, b2, W2r,
           Wlin1, blin1, Wlin2, blin2):
    del edge_weight
    f32 = jnp.float32
    nb = NP // RB

    xp = jnp.pad(x, ((0, NP - N), (0, 0)))
    src2 = jnp.pad(edge_index[0], (0, EPAD - E), constant_values=N).reshape(
        EPAD // 128, 128)
    dst2 = jnp.pad(edge_index[1], (0, EPAD - E), constant_values=N).reshape(
        EPAD // 128, 128)
    bat2 = jnp.pad(batch, (0, BATP - N), constant_values=G).reshape(
        BATP // 128, 128)

    x0 = xp[:, 0:32]
    x1 = xp[:, 32:64]
    x2 = xp[:, 64:80]
    x3 = jnp.pad(xp[:, 80:90], ((0, 0), (0, 6)))

    cntp, histp = _cnt_call()(dst2, bat2)
    agga = _agg_call(32)(src2, dst2, x0, x1)
    aggb = _agg_call(16)(src2, dst2, x2, x3)

    xt, h0, h1 = pl.pallas_call(
        _n1_body,
        grid=(nb,),
        in_specs=[_row_block((NP, 32)), _row_block((NP, 32)),
                  _row_block((NP, 16)), _row_block((NP, 16)),
                  _row_block((NP, 16)), _row_block((NP, 16)),
                  _row_block((NP, D)), _full((D, HC)), _full((D, HC)),
                  _full((1, HC))],
        out_specs=[_row_block((NP, HC)), _row_block((NP, 32)),
                   _row_block((NP, 32))],
        out_shape=[jax.ShapeDtypeStruct((NP, HC), f32),
                   jax.ShapeDtypeStruct((NP, 32), f32),
                   jax.ShapeDtypeStruct((NP, 32), f32)],
    )(agga[0], agga[1], aggb[0], aggb[1], cntp[0], cntp[1], xp, W1l, W1r,
      b1.reshape(1, HC))

    agg2 = _agg_call(32)(src2, dst2, h0, h1)

    h2, key = pl.pallas_call(
        _n2_body,
        grid=(nb,),
        in_specs=[_row_block((NP, 32)), _row_block((NP, 32)),
                  _row_block((NP, 16)), _row_block((NP, 16)),
                  _row_block((NP, HC)), _full((HC, HC)), _full((HC, HC)),
                  _full((1, HC))],
        out_specs=[_row_block((NP, HC)), _row_block((NP, 1))],
        out_shape=[jax.ShapeDtypeStruct((NP, HC), f32),
                   jax.ShapeDtypeStruct((NP, 1), f32)],
    )(agg2[0], agg2[1], cntp[0], cntp[1], xt, W2l, W2r, b2.reshape(1, HC))

    keyp = jnp.pad(key.reshape(NP), (0, KEYPAD - NP))
    pool = _pool_call()(keyp, h2, histp[0], histp[1])

    out = pl.pallas_call(
        _f_body,
        grid=(1,),
        in_specs=[_full((G, KTOP * HC)), _full((KTOP * HC, HC)),
                  _full((1, HC)), _full((HC, 1)), _full((1, 1))],
        out_specs=_full((G, 1)),
        out_shape=jax.ShapeDtypeStruct((G, 1), f32),
    )(pool.reshape(G, KTOP * HC), Wlin1, blin1.reshape(1, HC),
      Wlin2, blin2.reshape(1, 1))

    return (out.reshape(G), xt[:N])


# speedup vs baseline: 1.1656x; 1.1198x over previous
import functools

import jax
import jax.numpy as jnp
from jax import lax
from jax.experimental import pallas as pl
from jax.experimental.pallas import tpu as pltpu
from jax.experimental.pallas import tpu_sc as plsc

N = 50000
NP = 50176
D = 90
HC = 64
KTOP = 70
G = 64
GP = 80
E = 800000
CH = 128
SUP = 3
SUPA = 6
EPR = 6432
EPAD = (EPR + SUPA) * CH
BATP = 53248
RB = 512
KEYPAD = 67584
CB = 16384
RSEG = 112


@functools.lru_cache(maxsize=1)
def _sc_mesh():
    return plsc.VectorSubcoreMesh(core_axis_name="c", subcore_axis_name="s",
                                  num_cores=2, num_subcores=16)


def _zero_vmem2d(ref, rows, cols):
    z16 = jnp.zeros((16,), jnp.float32)
    def body(i, _):
        for cj in range(cols // 16):
            ref[i, pl.ds(cj * 16, 16)] = z16
        return 0
    lax.fori_loop(0, rows, body, 0)


def _agg_body(w_cols, src2, dst2, t0, t1, zer, out, acc,
              ixs0, ixd0, ixs1, ixd1, rows0, semg0, sems0, semi):
    c = lax.axis_index("c")
    s = lax.axis_index("s")
    base = s * (NP // 16)
    rpt = EPR // 16
    nsup = rpt // SUPA
    tile0 = s * rpt

    pltpu.sync_copy(zer.at[pl.ds(base, NP // 16), :],
                    acc.at[pl.ds(base, NP // 16), :])
    plsc.subcore_barrier()

    def load_idx(ixs, ixd, sup):
        r0 = tile0 + sup * SUPA
        pltpu.async_copy(src2.at[pl.ds(r0, SUPA)], ixs, semi)
        pltpu.async_copy(dst2.at[pl.ds(r0, SUPA)], ixd, semi)

    def wait_idx(ixs, ixd, sup):
        r0 = tile0 + sup * SUPA
        pltpu.make_async_copy(src2.at[pl.ds(r0, SUPA)], ixs, semi).wait()
        pltpu.make_async_copy(dst2.at[pl.ds(r0, SUPA)], ixd, semi).wait()

    def run(tbl):
        load_idx(ixs0, ixd0, jnp.int32(0))

        def one_sup(i, ixs, ixd, ixsn, ixdn):
            wait_idx(ixs, ixd, i)
            load_idx(ixsn, ixdn, i + 1)
            cps = [pltpu.async_copy(tbl.at[ixs.at[j]], rows0.at[j], semg0)
                   for j in range(SUPA)]
            for cp in cps:
                cp.wait()
            cps = [pltpu.async_copy(rows0.at[j], acc.at[ixd.at[j]], sems0,
                                    add=True) for j in range(SUPA)]
            for cp in cps:
                cp.wait()

        def pair_body(i, _):
            one_sup(2 * i, ixs0, ixd0, ixs1, ixd1)
            one_sup(2 * i + 1, ixs1, ixd1, ixs0, ixd0)
            return 0
        lax.fori_loop(0, nsup // 2, pair_body, 0)
        one_sup(jnp.int32(nsup - 1), ixs0, ixd0, ixs1, ixd1)
        wait_idx(ixs1, ixd1, jnp.int32(nsup))

    @pl.when(c == 0)
    def _():
        run(t0)

    @pl.when(c == 1)
    def _():
        run(t1)

    plsc.subcore_barrier()
    pltpu.sync_copy(acc.at[pl.ds(base, NP // 16), :],
                    out.at[c, pl.ds(base, NP // 16), :])


@functools.lru_cache(maxsize=None)
def _agg_call(w_cols):
    return pl.kernel(
        functools.partial(_agg_body, w_cols),
        out_type=jax.ShapeDtypeStruct((2, NP, w_cols), jnp.float32),
        mesh=_sc_mesh(),
        compiler_params=pltpu.CompilerParams(use_tc_tiling_on_sc=False,
                                             needs_layout_passes=False),
        scratch_types=[
            pltpu.VMEM_SHARED((NP, w_cols), jnp.float32),
            pltpu.VMEM((SUPA, CH), jnp.int32),
            pltpu.VMEM((SUPA, CH), jnp.int32),
            pltpu.VMEM((SUPA, CH), jnp.int32),
            pltpu.VMEM((SUPA, CH), jnp.int32),
            pltpu.VMEM((SUPA, CH, w_cols), jnp.float32),
            pltpu.SemaphoreType.DMA,
            pltpu.SemaphoreType.DMA,
            pltpu.SemaphoreType.DMA,
        ],
    )


def _cnt_body(dst2, bat2, zer, outc, outg, accc, accg, idx_d, ones_v, sem):
    c = lax.axis_index("c")
    s = lax.axis_index("s")
    w = c * 16 + s
    base = s * (NP // 16)

    one16 = jnp.ones((16,), jnp.float32)
    def fill_ones(i, _):
        ones_v[i, pl.ds(0, 16)] = one16
        return 0
    lax.fori_loop(0, CH, fill_ones, 0)

    pltpu.sync_copy(zer.at[pl.ds(base, NP // 16), :],
                    accc.at[pl.ds(base, NP // 16), :])
    @pl.when(s == 0)
    def _():
        pltpu.sync_copy(zer.at[pl.ds(0, GP), :], accg)
    plsc.subcore_barrier()

    def cnt_sc(i, _):
        row0 = w * (EPR // 32) + i * SUP
        pltpu.sync_copy(dst2.at[pl.ds(row0, SUP)], idx_d)
        for j in range(SUP):
            pltpu.sync_copy(ones_v, accc.at[idx_d.at[j]], add=True)
        return 0
    lax.fori_loop(0, EPR // 32 // SUP, cnt_sc, 0)

    def bat_sc(i, _):
        row0 = w * (BATP // 128 // 32) + i
        pltpu.sync_copy(bat2.at[pl.ds(row0, 1)], idx_d.at[pl.ds(0, 1)])
        pltpu.sync_copy(ones_v, accg.at[idx_d.at[0]], add=True)
        return 0
    lax.fori_loop(0, BATP // 128 // 32, bat_sc, 0)

    plsc.subcore_barrier()
    pltpu.sync_copy(accc.at[pl.ds(base, NP // 16), :],
                    outc.at[c, pl.ds(base, NP // 16), :])
    @pl.when(s == 0)
    def _():
        pltpu.sync_copy(accg, outg.at[c])


@functools.lru_cache(maxsize=1)
def _cnt_call():
    return pl.kernel(
        _cnt_body,
        out_type=(jax.ShapeDtypeStruct((2, NP, 16), jnp.float32),
                  jax.ShapeDtypeStruct((2, GP, 16), jnp.float32)),
        mesh=_sc_mesh(),
        compiler_params=pltpu.CompilerParams(use_tc_tiling_on_sc=False,
                                             needs_layout_passes=False),
        scratch_types=[
            pltpu.VMEM_SHARED((NP, 16), jnp.float32),
            pltpu.VMEM_SHARED((GP, 16), jnp.float32),
            pltpu.VMEM((SUP, CH), jnp.int32),
            pltpu.VMEM((CH, 16), jnp.float32),
            pltpu.SemaphoreType.DMA,
        ],
    )


def _pool_body(keyp, h2, hg0, hg1, pool, keys_v, rows_v, sel_v, h0_v, h1_v,
               st_s, cn_s, sem):
    c = lax.axis_index("c")
    s = lax.axis_index("s")
    w = c * 16 + s

    pltpu.sync_copy(hg0, h0_v)
    pltpu.sync_copy(hg1, h1_v)

    lanes = lax.iota(jnp.int32, 16)
    NEG = jnp.float32(-jnp.inf)
    BIG = jnp.int32(2**31 - 1)

    run = jnp.int32(0)
    for g in range(G):
        cg = (h0_v[g, pl.ds(0, 16)][0]
              + h1_v[g, pl.ds(0, 16)][0]).astype(jnp.int32)
        st_s[g] = run
        cn_s[g] = cg
        run = run + cg

    z16i = jnp.zeros((16,), jnp.int32)
    for o in (0, 16, 32, 48, KTOP + 2 - 16):
        sel_v[pl.ds(o, 16)] = z16i

    for gi in range(2):
        g = w * 2 + gi
        start = st_s[g]
        cnt = cn_s[g]
        abase = (start // 8) * 8
        off = start - abase
        nch = (cnt + (CB - 1)) // CB

        def load_chunk(ch):
            pltpu.sync_copy(keyp.at[pl.ds(abase + ch * CB, CB + 16)], keys_v)

        @pl.when(nch <= 1)
        def _():
            load_chunk(jnp.int32(0))

        def round_body(r, th):
            tk, ti = th

            def chunk_body(ch, bb):
                bk, bi = bb
                @pl.when(nch > 1)
                def _():
                    load_chunk(ch)
                glend = jnp.minimum((ch + 1) * CB, cnt)
                jtrip = (off + (glend - ch * CB) + 15) // 16

                def scan_body(j, bb2):
                    bk2, bi2 = bb2
                    k16 = keys_v[pl.ds(j * 16, 16)]
                    gl = ch * CB + j * 16 + lanes - off
                    ok = (gl >= ch * CB) & (gl < glend)
                    elig = ok & ((k16 < tk) | ((k16 == tk) & (gl > ti)))
                    better = (k16 > bk2) | ((k16 == bk2) & (gl < bi2))
                    upd = elig & better
                    return (jnp.where(upd, k16, bk2), jnp.where(upd, gl, bi2))

                return lax.fori_loop(0, jtrip, scan_body, (bk, bi))

            bk0 = jnp.full((16,), NEG, jnp.float32)
            bi0 = jnp.full((16,), BIG, jnp.int32)
            bk, bi = lax.fori_loop(0, nch, chunk_body, (bk0, bi0))
            m = jnp.max(bk)
            il = jnp.min(jnp.where(bk == m, bi, BIG))
            val = jnp.where(r < cnt, start + il, 0) + z16i
            plsc.store_scatter(sel_v, [r + z16i], val, mask=lanes == 0)
            return (m, il)

        lax.fori_loop(0, KTOP, round_body, (jnp.float32(jnp.inf),
                                            jnp.int32(-1)))

        pltpu.async_copy(h2.at[sel_v], rows_v, sem).wait()
        zz = jnp.zeros((16,), jnp.float32)
        def mask_body(r, _):
            ok = r < cnt
            for cj in range(HC // 16):
                v = rows_v[r, pl.ds(cj * 16, 16)]
                rows_v[r, pl.ds(cj * 16, 16)] = jnp.where(ok, v, zz)
            return 0
        lax.fori_loop(0, KTOP, mask_body, 0)
        pltpu.sync_copy(rows_v.at[pl.ds(0, KTOP), :], pool.at[g])


@functools.lru_cache(maxsize=1)
def _pool_call():
    return pl.kernel(
        _pool_body,
        out_type=jax.ShapeDtypeStruct((G, KTOP, HC), jnp.float32),
        mesh=_sc_mesh(),
        compiler_params=pltpu.CompilerParams(use_tc_tiling_on_sc=False,
                                             needs_layout_passes=False),
        scratch_types=[
            pltpu.VMEM((CB + 16,), jnp.float32),
            pltpu.VMEM((KTOP + 2, HC), jnp.float32),
            pltpu.VMEM((KTOP + 2,), jnp.int32),
            pltpu.VMEM((GP, 16), jnp.float32),
            pltpu.VMEM((GP, 16), jnp.float32),
            pltpu.SMEM((GP,), jnp.int32),
            pltpu.SMEM((GP,), jnp.int32),
            pltpu.SemaphoreType.DMA,
        ],
    )


def _n1_body(a0_ref, a1_ref, a2_ref, a3_ref, c0_ref, c1_ref, x_ref, wl_ref,
             wr_ref, b1_ref, xt_ref, h0_ref, h1_ref):
    cnt = jnp.maximum(c0_ref[:, 0:1] + c1_ref[:, 0:1], 1.0)
    mean = jnp.concatenate(
        [a0_ref[...], a1_ref[...], a2_ref[...], a3_ref[...]], axis=1)[:, :D]
    mean = mean / cnt
    xt = (jnp.dot(mean, wl_ref[...], preferred_element_type=jnp.float32)
          + b1_ref[...]) + jnp.dot(x_ref[...], wr_ref[...],
                                   preferred_element_type=jnp.float32)
    xt_ref[...] = xt
    h = jnp.maximum(xt, 0.0)
    h0_ref[...] = h[:, :32]
    h1_ref[...] = h[:, 32:]


def _n2_body(a0_ref, a1_ref, c0_ref, c1_ref, xt_ref, wl_ref, wr_ref, b2_ref,
             h2_ref, key_ref):
    cnt = jnp.maximum(c0_ref[:, 0:1] + c1_ref[:, 0:1], 1.0)
    mean = jnp.concatenate([a0_ref[...], a1_ref[...]], axis=1) / cnt
    h = jnp.maximum(xt_ref[...], 0.0)
    h2 = (jnp.dot(mean, wl_ref[...], preferred_element_type=jnp.float32)
          + b2_ref[...]) + jnp.dot(h, wr_ref[...],
                                   preferred_element_type=jnp.float32)
    h2_ref[...] = h2
    key_ref[...] = h2[:, HC - 1:HC]


def _f_body(p_ref, w1_ref, b1_ref, w2_ref, b2_ref, o_ref):
    p1 = (jnp.dot(p_ref[...], w1_ref[...], preferred_element_type=jnp.float32)
          + b1_ref[...])
    p2 = (jnp.dot(p1, w2_ref[...], preferred_element_type=jnp.float32)
          + b2_ref[...])
    o_ref[...] = jax.nn.sigmoid(p2)


def _row_block(shape):
    return pl.BlockSpec((RB,) + shape[1:],
                        lambda i: (i,) + (0,) * (len(shape) - 1))


def _full(shape):
    return pl.BlockSpec(shape, lambda i: (0,) * len(shape))


def kernel(x, edge_index, edge_weight, batch, W1l, b1, W1r, W2l, b2, W2r,
           Wlin1, blin1, Wlin2, blin2):
    del edge_weight
    f32 = jnp.float32
    nb = NP // RB

    xp = jnp.pad(x, ((0, NP - N), (0, 0)))
    src2 = jnp.pad(edge_index[0], (0, EPAD - E), constant_values=N).reshape(
        EPAD // 128, 128)
    dst2 = jnp.pad(edge_index[1], (0, EPAD - E), constant_values=N).reshape(
        EPAD // 128, 128)
    bat2 = jnp.pad(batch, (0, BATP - N), constant_values=G).reshape(
        BATP // 128, 128)

    x0 = xp[:, 0:32]
    x1 = xp[:, 32:64]
    x2 = xp[:, 64:80]
    x3 = jnp.pad(xp[:, 80:90], ((0, 0), (0, 6)))

    z16 = jnp.zeros((NP, 16), f32)
    z32 = jnp.zeros((NP, 32), f32)
    cntp, histp = _cnt_call()(dst2, bat2, z16)
    agga = _agg_call(32)(src2, dst2, x0, x1, z32)
    aggb = _agg_call(16)(src2, dst2, x2, x3, z16)

    xt, h0, h1 = pl.pallas_call(
        _n1_body,
        grid=(nb,),
        in_specs=[_row_block((NP, 32)), _row_block((NP, 32)),
                  _row_block((NP, 16)), _row_block((NP, 16)),
                  _row_block((NP, 16)), _row_block((NP, 16)),
                  _row_block((NP, D)), _full((D, HC)), _full((D, HC)),
                  _full((1, HC))],
        out_specs=[_row_block((NP, HC)), _row_block((NP, 32)),
                   _row_block((NP, 32))],
        out_shape=[jax.ShapeDtypeStruct((NP, HC), f32),
                   jax.ShapeDtypeStruct((NP, 32), f32),
                   jax.ShapeDtypeStruct((NP, 32), f32)],
    )(agga[0], agga[1], aggb[0], aggb[1], cntp[0], cntp[1], xp, W1l, W1r,
      b1.reshape(1, HC))

    agg2 = _agg_call(32)(src2, dst2, h0, h1, z32)

    h2, key = pl.pallas_call(
        _n2_body,
        grid=(nb,),
        in_specs=[_row_block((NP, 32)), _row_block((NP, 32)),
                  _row_block((NP, 16)), _row_block((NP, 16)),
                  _row_block((NP, HC)), _full((HC, HC)), _full((HC, HC)),
                  _full((1, HC))],
        out_specs=[_row_block((NP, HC)), _row_block((NP, 1))],
        out_shape=[jax.ShapeDtypeStruct((NP, HC), f32),
                   jax.ShapeDtypeStruct((NP, 1), f32)],
    )(agg2[0], agg2[1], cntp[0], cntp[1], xt, W2l, W2r, b2.reshape(1, HC))

    keyp = jnp.pad(key.reshape(NP), (0, KEYPAD - NP))
    pool = _pool_call()(keyp, h2, histp[0], histp[1])

    out = pl.pallas_call(
        _f_body,
        grid=(1,),
        in_specs=[_full((G, KTOP * HC)), _full((KTOP * HC, HC)),
                  _full((1, HC)), _full((HC, 1)), _full((1, 1))],
        out_specs=_full((G, 1)),
        out_shape=jax.ShapeDtypeStruct((G, 1), f32),
    )(pool.reshape(G, KTOP * HC), Wlin1, blin1.reshape(1, HC),
      Wlin2, blin2.reshape(1, 1))

    return (out.reshape(G), xt[:N])


# speedup vs baseline: 1.2307x; 1.0558x over previous
import functools

import jax
import jax.numpy as jnp
from jax import lax
from jax.experimental import pallas as pl
from jax.experimental.pallas import tpu as pltpu
from jax.experimental.pallas import tpu_sc as plsc

N = 50000
NP = 50176
D = 90
HC = 64
KTOP = 70
G = 64
GP = 80
E = 800000
CH = 128
SUP = 3
SUPA = 6
EPR = 6432
EPAD = (EPR + SUPA) * CH
BATP = 53248
RB = 512
KEYPAD = 67584
CB = 16384
RSEG = 112


@functools.lru_cache(maxsize=1)
def _sc_mesh():
    return plsc.VectorSubcoreMesh(core_axis_name="c", subcore_axis_name="s",
                                  num_cores=2, num_subcores=16)


def _zero_vmem2d(ref, rows, cols):
    z16 = jnp.zeros((16,), jnp.float32)
    def body(i, _):
        for cj in range(cols // 16):
            ref[i, pl.ds(cj * 16, 16)] = z16
        return 0
    lax.fori_loop(0, rows, body, 0)


def _agg_body(w_cols, src2, dst2, t0, t1, zer, out, acc,
              ixs0, ixd0, ixs1, ixd1, rows0, semg0, sems0, semi):
    c = lax.axis_index("c")
    s = lax.axis_index("s")
    base = s * (NP // 16)
    rpt = EPR // 16
    nsup = rpt // SUPA
    tile0 = s * rpt

    pltpu.sync_copy(zer.at[pl.ds(base, NP // 16), :],
                    acc.at[pl.ds(base, NP // 16), :])
    plsc.subcore_barrier()

    def load_idx(ixs, ixd, sup):
        r0 = tile0 + sup * SUPA
        pltpu.async_copy(src2.at[pl.ds(r0, SUPA)], ixs, semi)
        pltpu.async_copy(dst2.at[pl.ds(r0, SUPA)], ixd, semi)

    def wait_idx(ixs, ixd, sup):
        r0 = tile0 + sup * SUPA
        pltpu.make_async_copy(src2.at[pl.ds(r0, SUPA)], ixs, semi).wait()
        pltpu.make_async_copy(dst2.at[pl.ds(r0, SUPA)], ixd, semi).wait()

    def run(tbl):
        load_idx(ixs0, ixd0, jnp.int32(0))

        def one_sup(i, ixs, ixd, ixsn, ixdn):
            wait_idx(ixs, ixd, i)
            load_idx(ixsn, ixdn, i + 1)
            gcps = [pltpu.async_copy(tbl.at[ixs.at[j]], rows0.at[j], semg0)
                    for j in range(SUPA)]
            scps = []
            for j in range(SUPA):
                gcps[j].wait()
                scps.append(pltpu.async_copy(rows0.at[j], acc.at[ixd.at[j]],
                                             sems0, add=True))
            for cp in scps:
                cp.wait()

        def pair_body(i, _):
            one_sup(2 * i, ixs0, ixd0, ixs1, ixd1)
            one_sup(2 * i + 1, ixs1, ixd1, ixs0, ixd0)
            return 0
        lax.fori_loop(0, nsup // 2, pair_body, 0)
        one_sup(jnp.int32(nsup - 1), ixs0, ixd0, ixs1, ixd1)
        wait_idx(ixs1, ixd1, jnp.int32(nsup))

    @pl.when(c == 0)
    def _():
        run(t0)

    @pl.when(c == 1)
    def _():
        run(t1)

    plsc.subcore_barrier()
    pltpu.sync_copy(acc.at[pl.ds(base, NP // 16), :],
                    out.at[c, pl.ds(base, NP // 16), :])


@functools.lru_cache(maxsize=None)
def _agg_call(w_cols):
    return pl.kernel(
        functools.partial(_agg_body, w_cols),
        out_type=jax.ShapeDtypeStruct((2, NP, w_cols), jnp.float32),
        mesh=_sc_mesh(),
        compiler_params=pltpu.CompilerParams(use_tc_tiling_on_sc=False,
                                             needs_layout_passes=False),
        scratch_types=[
            pltpu.VMEM_SHARED((NP, w_cols), jnp.float32),
            pltpu.VMEM((SUPA, CH), jnp.int32),
            pltpu.VMEM((SUPA, CH), jnp.int32),
            pltpu.VMEM((SUPA, CH), jnp.int32),
            pltpu.VMEM((SUPA, CH), jnp.int32),
            pltpu.VMEM((SUPA, CH, w_cols), jnp.float32),
            pltpu.SemaphoreType.DMA,
            pltpu.SemaphoreType.DMA,
            pltpu.SemaphoreType.DMA,
        ],
    )


def _aggc_body(src2, dst2, bat2, t0, t1, zer, out, outc, outg,
               acc, accc, accg, ixs0, ixd0, ixs1, ixd1, rows0, ones_v,
               semg0, sems0, semi):
    c = lax.axis_index("c")
    s = lax.axis_index("s")
    w = c * 16 + s
    base = s * (NP // 16)
    rpt = EPR // 16
    nsup = rpt // SUPA
    tile0 = s * rpt

    one16 = jnp.ones((16,), jnp.float32)
    def fill_ones(i, _):
        ones_v[i, pl.ds(0, 16)] = one16
        return 0
    lax.fori_loop(0, CH, fill_ones, 0)

    pltpu.sync_copy(zer.at[pl.ds(base, NP // 16), :],
                    acc.at[pl.ds(base, NP // 16), :])
    pltpu.sync_copy(zer.at[pl.ds(base, NP // 16), :],
                    accc.at[pl.ds(base, NP // 16), :])
    @pl.when(s == 0)
    def _():
        pltpu.sync_copy(zer.at[pl.ds(0, GP), :], accg)
    plsc.subcore_barrier()

    def load_idx(ixs, ixd, sup):
        r0 = tile0 + sup * SUPA
        pltpu.async_copy(src2.at[pl.ds(r0, SUPA)], ixs, semi)
        pltpu.async_copy(dst2.at[pl.ds(r0, SUPA)], ixd, semi)

    def wait_idx(ixs, ixd, sup):
        r0 = tile0 + sup * SUPA
        pltpu.make_async_copy(src2.at[pl.ds(r0, SUPA)], ixs, semi).wait()
        pltpu.make_async_copy(dst2.at[pl.ds(r0, SUPA)], ixd, semi).wait()

    def run(tbl):
        load_idx(ixs0, ixd0, jnp.int32(0))

        def one_sup(i, ixs, ixd, ixsn, ixdn):
            wait_idx(ixs, ixd, i)
            load_idx(ixsn, ixdn, i + 1)
            do_cnt = jnp.where(c == 0, i < (nsup + 1) // 2,
                               i >= (nsup + 1) // 2)
            gcps = [pltpu.async_copy(tbl.at[ixs.at[j]], rows0.at[j], semg0)
                    for j in range(SUPA)]
            scps = []
            @pl.when(do_cnt)
            def _():
                for j in range(SUPA):
                    pltpu.async_copy(ones_v, accc.at[ixd.at[j]], sems0,
                                     add=True)
            for j in range(SUPA):
                gcps[j].wait()
                scps.append(pltpu.async_copy(rows0.at[j], acc.at[ixd.at[j]],
                                             sems0, add=True))
            for cp in scps:
                cp.wait()
            @pl.when(do_cnt)
            def _():
                for j in range(SUPA):
                    pltpu.make_async_copy(ones_v, accc.at[ixd.at[j]],
                                          sems0).wait()

        def pair_body(i, _):
            one_sup(2 * i, ixs0, ixd0, ixs1, ixd1)
            one_sup(2 * i + 1, ixs1, ixd1, ixs0, ixd0)
            return 0
        lax.fori_loop(0, nsup // 2, pair_body, 0)
        one_sup(jnp.int32(nsup - 1), ixs0, ixd0, ixs1, ixd1)
        wait_idx(ixs1, ixd1, jnp.int32(nsup))

    @pl.when(c == 0)
    def _():
        run(t0)

    @pl.when(c == 1)
    def _():
        run(t1)

    def bat_sc(i, _):
        row0 = w * (BATP // 128 // 32) + i
        pltpu.sync_copy(bat2.at[pl.ds(row0, 1)], ixs0.at[pl.ds(0, 1)])
        pltpu.sync_copy(ones_v, accg.at[ixs0.at[0]], add=True)
        return 0
    lax.fori_loop(0, BATP // 128 // 32, bat_sc, 0)

    plsc.subcore_barrier()
    pltpu.sync_copy(acc.at[pl.ds(base, NP // 16), :],
                    out.at[c, pl.ds(base, NP // 16), :])
    pltpu.sync_copy(accc.at[pl.ds(base, NP // 16), :],
                    outc.at[c, pl.ds(base, NP // 16), :])
    @pl.when(s == 0)
    def _():
        pltpu.sync_copy(accg, outg.at[c])


@functools.lru_cache(maxsize=1)
def _aggc_call():
    return pl.kernel(
        _aggc_body,
        out_type=(jax.ShapeDtypeStruct((2, NP, 16), jnp.float32),
                  jax.ShapeDtypeStruct((2, NP, 16), jnp.float32),
                  jax.ShapeDtypeStruct((2, GP, 16), jnp.float32)),
        mesh=_sc_mesh(),
        compiler_params=pltpu.CompilerParams(use_tc_tiling_on_sc=False,
                                             needs_layout_passes=False),
        scratch_types=[
            pltpu.VMEM_SHARED((NP, 16), jnp.float32),
            pltpu.VMEM_SHARED((NP, 16), jnp.float32),
            pltpu.VMEM_SHARED((GP, 16), jnp.float32),
            pltpu.VMEM((SUPA, CH), jnp.int32),
            pltpu.VMEM((SUPA, CH), jnp.int32),
            pltpu.VMEM((SUPA, CH), jnp.int32),
            pltpu.VMEM((SUPA, CH), jnp.int32),
            pltpu.VMEM((SUPA, CH, 16), jnp.float32),
            pltpu.VMEM((CH, 16), jnp.float32),
            pltpu.SemaphoreType.DMA,
            pltpu.SemaphoreType.DMA,
            pltpu.SemaphoreType.DMA,
        ],
    )


def _pool_body(keyp, h2, hg0, hg1, pool, keys_v, rows_v, sel_v, h0_v, h1_v,
               st_s, cn_s, sem):
    c = lax.axis_index("c")
    s = lax.axis_index("s")
    w = c * 16 + s

    pltpu.sync_copy(hg0, h0_v)
    pltpu.sync_copy(hg1, h1_v)

    lanes = lax.iota(jnp.int32, 16)
    NEG = jnp.float32(-jnp.inf)
    BIG = jnp.int32(2**31 - 1)

    run = jnp.int32(0)
    for g in range(G):
        cg = (h0_v[g, pl.ds(0, 16)][0]
              + h1_v[g, pl.ds(0, 16)][0]).astype(jnp.int32)
        st_s[g] = run
        cn_s[g] = cg
        run = run + cg

    z16i = jnp.zeros((16,), jnp.int32)
    for o in (0, 16, 32, 48, KTOP + 2 - 16):
        sel_v[pl.ds(o, 16)] = z16i

    for gi in range(2):
        g = w * 2 + gi
        start = st_s[g]
        cnt = cn_s[g]
        abase = (start // 8) * 8
        off = start - abase
        nch = (cnt + (CB - 1)) // CB

        def load_chunk(ch):
            pltpu.sync_copy(keyp.at[pl.ds(abase + ch * CB, CB + 16)], keys_v)

        @pl.when(nch <= 1)
        def _():
            load_chunk(jnp.int32(0))

        def round_body(r, th):
            tk, ti = th

            def chunk_body(ch, bb):
                bk, bi = bb
                @pl.when(nch > 1)
                def _():
                    load_chunk(ch)
                glend = jnp.minimum((ch + 1) * CB, cnt)
                jtrip = (off + (glend - ch * CB) + 15) // 16

                def scan_body(j, bb2):
                    bk2, bi2 = bb2
                    k16 = keys_v[pl.ds(j * 16, 16)]
                    gl = ch * CB + j * 16 + lanes - off
                    ok = (gl >= ch * CB) & (gl < glend)
                    elig = ok & ((k16 < tk) | ((k16 == tk) & (gl > ti)))
                    better = (k16 > bk2) | ((k16 == bk2) & (gl < bi2))
                    upd = elig & better
                    return (jnp.where(upd, k16, bk2), jnp.where(upd, gl, bi2))

                return lax.fori_loop(0, jtrip, scan_body, (bk, bi))

            bk0 = jnp.full((16,), NEG, jnp.float32)
            bi0 = jnp.full((16,), BIG, jnp.int32)
            bk, bi = lax.fori_loop(0, nch, chunk_body, (bk0, bi0))
            m = jnp.max(bk)
            il = jnp.min(jnp.where(bk == m, bi, BIG))
            val = jnp.where(r < cnt, start + il, 0) + z16i
            plsc.store_scatter(sel_v, [r + z16i], val, mask=lanes == 0)
            return (m, il)

        lax.fori_loop(0, KTOP, round_body, (jnp.float32(jnp.inf),
                                            jnp.int32(-1)))

        pltpu.async_copy(h2.at[sel_v], rows_v, sem).wait()
        zz = jnp.zeros((16,), jnp.float32)
        def mask_body(r, _):
            ok = r < cnt
            for cj in range(HC // 16):
                v = rows_v[r, pl.ds(cj * 16, 16)]
                rows_v[r, pl.ds(cj * 16, 16)] = jnp.where(ok, v, zz)
            return 0
        lax.fori_loop(0, KTOP, mask_body, 0)
        pltpu.sync_copy(rows_v.at[pl.ds(0, KTOP), :], pool.at[g])


@functools.lru_cache(maxsize=1)
def _pool_call():
    return pl.kernel(
        _pool_body,
        out_type=jax.ShapeDtypeStruct((G, KTOP, HC), jnp.float32),
        mesh=_sc_mesh(),
        compiler_params=pltpu.CompilerParams(use_tc_tiling_on_sc=False,
                                             needs_layout_passes=False),
        scratch_types=[
            pltpu.VMEM((CB + 16,), jnp.float32),
            pltpu.VMEM((KTOP + 2, HC), jnp.float32),
            pltpu.VMEM((KTOP + 2,), jnp.int32),
            pltpu.VMEM((GP, 16), jnp.float32),
            pltpu.VMEM((GP, 16), jnp.float32),
            pltpu.SMEM((GP,), jnp.int32),
            pltpu.SMEM((GP,), jnp.int32),
            pltpu.SemaphoreType.DMA,
        ],
    )


def _n1_body(a0_ref, a1_ref, a2_ref, a3_ref, c0_ref, c1_ref, x_ref, wl_ref,
             wr_ref, b1_ref, xt_ref, h0_ref, h1_ref):
    cnt = jnp.maximum(c0_ref[:, 0:1] + c1_ref[:, 0:1], 1.0)
    mean = jnp.concatenate(
        [a0_ref[...], a1_ref[...], a2_ref[...], a3_ref[...]], axis=1)[:, :D]
    mean = mean / cnt
    xt = (jnp.dot(mean, wl_ref[...], preferred_element_type=jnp.float32)
          + b1_ref[...]) + jnp.dot(x_ref[...], wr_ref[...],
                                   preferred_element_type=jnp.float32)
    xt_ref[...] = xt
    h = jnp.maximum(xt, 0.0)
    h0_ref[...] = h[:, :32]
    h1_ref[...] = h[:, 32:]


def _n2_body(a0_ref, a1_ref, c0_ref, c1_ref, xt_ref, wl_ref, wr_ref, b2_ref,
             h2_ref, key_ref):
    cnt = jnp.maximum(c0_ref[:, 0:1] + c1_ref[:, 0:1], 1.0)
    mean = jnp.concatenate([a0_ref[...], a1_ref[...]], axis=1) / cnt
    h = jnp.maximum(xt_ref[...], 0.0)
    h2 = (jnp.dot(mean, wl_ref[...], preferred_element_type=jnp.float32)
          + b2_ref[...]) + jnp.dot(h, wr_ref[...],
                                   preferred_element_type=jnp.float32)
    h2_ref[...] = h2
    key_ref[...] = h2[:, HC - 1:HC]


def _f_body(p_ref, w1_ref, b1_ref, w2_ref, b2_ref, o_ref):
    p1 = (jnp.dot(p_ref[...], w1_ref[...], preferred_element_type=jnp.float32)
          + b1_ref[...])
    p2 = (jnp.dot(p1, w2_ref[...], preferred_element_type=jnp.float32)
          + b2_ref[...])
    o_ref[...] = jax.nn.sigmoid(p2)


def _row_block(shape):
    return pl.BlockSpec((RB,) + shape[1:],
                        lambda i: (i,) + (0,) * (len(shape) - 1))


def _full(shape):
    return pl.BlockSpec(shape, lambda i: (0,) * len(shape))


def kernel(x, edge_index, edge_weight, batch, W1l, b1, W1r, W2l, b2, W2r,
           Wlin1, blin1, Wlin2, blin2):
    del edge_weight
    f32 = jnp.float32
    nb = NP // RB

    xp = jnp.pad(x, ((0, NP - N), (0, 0)))
    src2 = jnp.pad(edge_index[0], (0, EPAD - E), constant_values=N).reshape(
        EPAD // 128, 128)
    dst2 = jnp.pad(edge_index[1], (0, EPAD - E), constant_values=N).reshape(
        EPAD // 128, 128)
    bat2 = jnp.pad(batch, (0, BATP - N), constant_values=G).reshape(
        BATP // 128, 128)

    x0 = xp[:, 0:32]
    x1 = xp[:, 32:64]
    x2 = xp[:, 64:80]
    x3 = jnp.pad(xp[:, 80:90], ((0, 0), (0, 6)))

    z16 = jnp.zeros((NP, 16), f32)
    z32 = jnp.zeros((NP, 32), f32)
    agga = _agg_call(32)(src2, dst2, x0, x1, z32)
    aggb, cntp, histp = _aggc_call()(src2, dst2, bat2, x2, x3, z16)

    xt, h0, h1 = pl.pallas_call(
        _n1_body,
        grid=(nb,),
        in_specs=[_row_block((NP, 32)), _row_block((NP, 32)),
                  _row_block((NP, 16)), _row_block((NP, 16)),
                  _row_block((NP, 16)), _row_block((NP, 16)),
                  _row_block((NP, D)), _full((D, HC)), _full((D, HC)),
                  _full((1, HC))],
        out_specs=[_row_block((NP, HC)), _row_block((NP, 32)),
                   _row_block((NP, 32))],
        out_shape=[jax.ShapeDtypeStruct((NP, HC), f32),
                   jax.ShapeDtypeStruct((NP, 32), f32),
                   jax.ShapeDtypeStruct((NP, 32), f32)],
    )(agga[0], agga[1], aggb[0], aggb[1], cntp[0], cntp[1], xp, W1l, W1r,
      b1.reshape(1, HC))

    agg2 = _agg_call(32)(src2, dst2, h0, h1, z32)

    h2, key = pl.pallas_call(
        _n2_body,
        grid=(nb,),
        in_specs=[_row_block((NP, 32)), _row_block((NP, 32)),
                  _row_block((NP, 16)), _row_block((NP, 16)),
                  _row_block((NP, HC)), _full((HC, HC)), _full((HC, HC)),
                  _full((1, HC))],
        out_specs=[_row_block((NP, HC)), _row_block((NP, 1))],
        out_shape=[jax.ShapeDtypeStruct((NP, HC), f32),
                   jax.ShapeDtypeStruct((NP, 1), f32)],
    )(agg2[0], agg2[1], cntp[0], cntp[1], xt, W2l, W2r, b2.reshape(1, HC))

    keyp = jnp.pad(key.reshape(NP), (0, KEYPAD - NP))
    pool = _pool_call()(keyp, h2, histp[0], histp[1])

    out = pl.pallas_call(
        _f_body,
        grid=(1,),
        in_specs=[_full((G, KTOP * HC)), _full((KTOP * HC, HC)),
                  _full((1, HC)), _full((HC, 1)), _full((1, 1))],
        out_specs=_full((G, 1)),
        out_shape=jax.ShapeDtypeStruct((G, 1), f32),
    )(pool.reshape(G, KTOP * HC), Wlin1, blin1.reshape(1, HC),
      Wlin2, blin2.reshape(1, 1))

    return (out.reshape(G), xt[:N])
